# baseline (speedup 1.0000x reference)
import functools

import jax
import jax.numpy as jnp
from jax import lax
from jax.experimental import pallas as pl
from jax.experimental.pallas import tpu as pltpu
from jax.experimental.pallas import tpu_sc as plsc

V, D, H, C = 1000000, 32, 100, 100
B, L = 4096, 200

NC, NS = 2, 16
NW = NC * NS
RPW = B // NW


def _make_cbow_sum():
    mesh = plsc.VectorSubcoreMesh(core_axis_name="c", subcore_axis_name="s")

    @functools.partial(
        pl.kernel,
        mesh=mesh,
        out_type=jax.ShapeDtypeStruct((B * D,), jnp.float32),
        scratch_types=[
            pltpu.VMEM((L,), jnp.int32),
            pltpu.VMEM((L,), jnp.int32),
            pltpu.VMEM((L, D), jnp.float32),
            pltpu.VMEM((L, D), jnp.float32),
            pltpu.VMEM((RPW * D,), jnp.float32),
            pltpu.SemaphoreType.DMA,
            pltpu.SemaphoreType.DMA,
            pltpu.SemaphoreType.DMA,
            pltpu.SemaphoreType.DMA,
        ],
    )
    def cbow_sum(embed_hbm, x_hbm, out_hbm,
                 idx0, idx1, buf0, buf1, out_v, gsem0, gsem1, isem0, isem1):
        wid = lax.axis_index("s") * NC + lax.axis_index("c")
        base = wid * RPW
        idx = (idx0, idx1)
        buf = (buf0, buf1)
        gsem = (gsem0, gsem1)
        isem = (isem0, isem1)

        def fetch_idx(r, p):
            pltpu.make_async_copy(x_hbm.at[pl.ds((base + r) * L, L)],
                                  idx[p], isem[p]).start()

        def enqueue_row(p):
            def enq16(g, _):
                v = idx[p][pl.ds(g * 16, 16)]
                for q in range(16):
                    pltpu.make_async_copy(
                        embed_hbm.at[v[q]], buf[p].at[g * 16 + q],
                        gsem[p]).start()
                return 0

            lax.fori_loop(0, L // 16, enq16, 0)
            vt = idx[p][pl.ds(L - 16, 16)]
            for q in range(16 - (L - (L // 16) * 16), 16):
                pltpu.make_async_copy(
                    embed_hbm.at[vt[q]], buf[p].at[L - 16 + q],
                    gsem[p]).start()

        def drain_row(p):
            pltpu.make_async_copy(embed_hbm.at[pl.ds(0, L)],
                                  buf[p], gsem[p]).wait()

        def sum_rows(b, r):
            def acc(j, carry):
                a = list(carry)
                for q in range(4):
                    a[q] = a[q] + b[4 * j + q, 0:16]
                    a[4 + q] = a[4 + q] + b[4 * j + q, 16:32]
                return tuple(a)

            z = tuple(jnp.zeros((16,), jnp.float32) for _ in range(8))
            a = lax.fori_loop(0, L // 4, acc, z, unroll=5)
            out_v[pl.ds(r * D, 16)] = (a[0] + a[1]) + (a[2] + a[3])
            out_v[pl.ds(r * D + 16, 16)] = (a[4] + a[5]) + (a[6] + a[7])

        fetch_idx(0, 0)
        fetch_idx(1, 1)

        def step(r, _):
            def body(p):
                pltpu.make_async_copy(x_hbm.at[pl.ds((base + r) * L, L)],
                                      idx[p], isem[p]).wait()
                enqueue_row(p)

                @pl.when(r + 2 < RPW)
                def _():
                    fetch_idx(r + 2, p)

                @pl.when(r > 0)
                def _():
                    drain_row(1 - p)
                    sum_rows(buf[1 - p], r - 1)

            lax.cond(lax.rem(r, 2) == 0, lambda: body(0), lambda: body(1))
            return 0

        lax.fori_loop(0, RPW, step, 0)
        lp = (RPW - 1) % 2
        drain_row(lp)
        sum_rows(buf[lp], RPW - 1)
        pltpu.sync_copy(out_v, out_hbm.at[pl.ds(base * D, RPW * D)])

    return cbow_sum


_cbow_sum = _make_cbow_sum()


_TCHUNK = 8192


def _t_body(in_ref, out_ref):
    out_ref[...] = lax.dot_general(
        in_ref[...], jnp.eye(32, dtype=jnp.float32),
        (((0,), (0,)), ((), ())),
        preferred_element_type=jnp.float32,
        precision=lax.Precision.HIGHEST,
    )


def _transpose_table(embed):
    grid = (V + _TCHUNK - 1) // _TCHUNK
    return pl.pallas_call(
        _t_body,
        grid=(grid,),
        in_specs=[pl.BlockSpec((D, _TCHUNK), lambda i: (0, i))],
        out_specs=pl.BlockSpec((_TCHUNK, D), lambda i: (i, 0)),
        out_shape=jax.ShapeDtypeStruct((V, D), jnp.float32),
    )(embed.T)


def _mlp_body(s_ref, w1_ref, b1_ref, w2_ref, b2_ref, out_ref):
    h = jnp.dot(s_ref[...], w1_ref[...], preferred_element_type=jnp.float32)
    h = jnp.maximum(h + b1_ref[...], 0.0)
    out_ref[...] = (
        jnp.dot(h, w2_ref[...], preferred_element_type=jnp.float32) + b2_ref[...]
    )


def kernel(x, embed, W1, b1, W2, b2):
    s = _cbow_sum(_transpose_table(embed), x.reshape(B * L)).reshape(B, D)
    out = pl.pallas_call(
        _mlp_body,
        out_shape=jax.ShapeDtypeStruct((B, C), jnp.float32),
    )(s, W1, b1.reshape(1, H), W2, b2.reshape(1, C))
    return out


# speedup vs baseline: 1.3370x; 1.3370x over previous
import functools

import jax
import jax.numpy as jnp
from jax import lax
from jax.experimental import pallas as pl
from jax.experimental.pallas import tpu as pltpu
from jax.experimental.pallas import tpu_sc as plsc

V, D, H, C = 1000000, 32, 100, 100
B, L = 4096, 200

NC, NS = 2, 16
NW = NC * NS
RPW = B // NW


def _make_cbow_sum():
    mesh = plsc.VectorSubcoreMesh(core_axis_name="c", subcore_axis_name="s")

    @functools.partial(
        pl.kernel,
        mesh=mesh,
        out_type=jax.ShapeDtypeStruct((B * D,), jnp.float32),
        scratch_types=[
            pltpu.VMEM((L,), jnp.int32),
            pltpu.VMEM((L,), jnp.int32),
            pltpu.VMEM((L, D), jnp.float32),
            pltpu.VMEM((L, D), jnp.float32),
            pltpu.VMEM((RPW * D,), jnp.float32),
            pltpu.SemaphoreType.DMA,
            pltpu.SemaphoreType.DMA,
            pltpu.SemaphoreType.DMA,
            pltpu.SemaphoreType.DMA,
        ],
    )
    def cbow_sum(embed_hbm, x_hbm, out_hbm,
                 idx0, idx1, buf0, buf1, out_v, gsem0, gsem1, isem0, isem1):
        wid = lax.axis_index("s") * NC + lax.axis_index("c")
        base = wid * RPW
        idx = (idx0, idx1)
        buf = (buf0, buf1)
        gsem = (gsem0, gsem1)
        isem = (isem0, isem1)

        def fetch_idx(r, p):
            pltpu.make_async_copy(x_hbm.at[pl.ds((base + r) * L, L)],
                                  idx[p], isem[p]).start()

        def enqueue_row(p):
            def enq16(g, _):
                v = idx[p][pl.ds(g * 16, 16)]
                for q in range(16):
                    pltpu.make_async_copy(
                        embed_hbm.at[v[q]], buf[p].at[g * 16 + q],
                        gsem[p]).start()
                return 0

            lax.fori_loop(0, L // 16, enq16, 0)
            vt = idx[p][pl.ds(L - 16, 16)]
            for q in range(16 - (L - (L // 16) * 16), 16):
                pltpu.make_async_copy(
                    embed_hbm.at[vt[q]], buf[p].at[L - 16 + q],
                    gsem[p]).start()

        def drain_row(p):
            pltpu.make_async_copy(embed_hbm.at[pl.ds(0, L)],
                                  buf[p], gsem[p]).wait()

        def sum_rows(b, r):
            def acc(j, carry):
                a = list(carry)
                for q in range(4):
                    a[q] = a[q] + b[4 * j + q, 0:16]
                    a[4 + q] = a[4 + q] + b[4 * j + q, 16:32]
                return tuple(a)

            z = tuple(jnp.zeros((16,), jnp.float32) for _ in range(8))
            a = lax.fori_loop(0, L // 4, acc, z, unroll=5)
            out_v[pl.ds(r * D, 16)] = (a[0] + a[1]) + (a[2] + a[3])
            out_v[pl.ds(r * D + 16, 16)] = (a[4] + a[5]) + (a[6] + a[7])

        fetch_idx(0, 0)
        fetch_idx(1, 1)

        def step(r, _):
            def body(p):
                pltpu.make_async_copy(x_hbm.at[pl.ds((base + r) * L, L)],
                                      idx[p], isem[p]).wait()
                enqueue_row(p)

                @pl.when(r + 2 < RPW)
                def _():
                    fetch_idx(r + 2, p)

                @pl.when(r > 0)
                def _():
                    drain_row(1 - p)
                    sum_rows(buf[1 - p], r - 1)

            lax.cond(lax.rem(r, 2) == 0, lambda: body(0), lambda: body(1))
            return 0

        lax.fori_loop(0, RPW, step, 0)
        lp = (RPW - 1) % 2
        drain_row(lp)
        sum_rows(buf[lp], RPW - 1)
        pltpu.sync_copy(out_v, out_hbm.at[pl.ds(base * D, RPW * D)])

    return cbow_sum


_cbow_sum = _make_cbow_sum()


def _mlp_body(s_ref, w1_ref, b1_ref, w2_ref, b2_ref, out_ref):
    h = jnp.dot(s_ref[...], w1_ref[...], preferred_element_type=jnp.float32)
    h = jnp.maximum(h + b1_ref[...], 0.0)
    out_ref[...] = (
        jnp.dot(h, w2_ref[...], preferred_element_type=jnp.float32) + b2_ref[...]
    )


def kernel(x, embed, W1, b1, W2, b2):
    s = _cbow_sum(embed, x.reshape(B * L)).reshape(B, D)
    out = pl.pallas_call(
        _mlp_body,
        out_shape=jax.ShapeDtypeStruct((B, C), jnp.float32),
    )(s, W1, b1.reshape(1, H), W2, b2.reshape(1, C))
    return out
